# baseline (speedup 1.0000x reference)
import jax
import jax.numpy as jnp
from jax.experimental import pallas as pl

S = 2048
D = 1024
H = 8
QD = 256
HALF = 128
NE = 256
TOPK = 8
KP = 8

T1 = 256
T2 = 128
TR = 128


def _qproj_kernel(hs_ref, wq_ref, bq_ref, dwn_ref, q_ref, hd_ref, stats_ref):
    i = pl.program_id(0)
    hs = hs_ref[...]
    q = jnp.dot(hs, wq_ref[...], preferred_element_type=jnp.float32) + bq_ref[...]
    q_ref[...] = q
    hd_ref[...] = jnp.dot(hs, dwn_ref[...], preferred_element_type=jnp.float32)
    s = jnp.sum(q, axis=0, keepdims=True)
    sq = jnp.sum(q * q, axis=0, keepdims=True)
    st = jnp.concatenate([s, sq], axis=0)

    @pl.when(i == 0)
    def _():
        stats_ref[...] = st

    @pl.when(i != 0)
    def _():
        stats_ref[...] = stats_ref[...] + st


def _l2rows(x):
    n = jnp.sqrt(jnp.sum(x * x, axis=-1, keepdims=True))
    return x / jnp.maximum(n, 1e-12)


def _topk_f32(vals, k):
    iota = jax.lax.broadcasted_iota(jnp.int32, vals.shape, 1)
    ss, ii = [], []
    for _ in range(k):
        m = jnp.max(vals, axis=-1, keepdims=True)
        idx = jnp.argmax(vals, axis=-1).astype(jnp.int32)[:, None]
        ss.append(m)
        ii.append(idx)
        vals = jnp.where(iota == idx, -1e30, vals)
    return jnp.concatenate(ss, axis=-1), jnp.concatenate(ii, axis=-1)


def _route_kernel(q_ref, hd_ref, mean_ref, rstd_ref, gam_ref, bet_ref,
                  sk0_ref, sk1_ref, up_ref, out_ref):
    R = q_ref.shape[0]
    sk0 = _l2rows(sk0_ref[...])
    sk1 = _l2rows(sk1_ref[...])
    dn = (((1,), (1,)), ((), ()))

    wt_parts = []
    for ti in range(R // TR):
        r0 = ti * TR
        qn = (q_ref[r0:r0 + TR, :] - mean_ref[...]) * rstd_ref[...] \
            * gam_ref[...] + bet_ref[...]
        qa = _l2rows(qn[:, :HALF])
        qb = _l2rows(qn[:, HALF:])
        ds0 = jax.lax.dot_general(qa, sk0, dn,
                                  preferred_element_type=jnp.float32)
        ds1 = jax.lax.dot_general(qb, sk1, dn,
                                  preferred_element_type=jnp.float32)

        s01, i01 = _topk_f32(jnp.concatenate([ds0, ds1], axis=0), KP)
        s0, s1 = s01[:TR], s01[TR:]
        i0, i1 = i01[:TR], i01[TR:]

        m64 = jax.lax.broadcasted_iota(jnp.int32, (TR, KP * KP), 1)
        cs = jnp.take_along_axis(s0, m64 >> 3, axis=1) \
            + jnp.take_along_axis(s1, m64 & 7, axis=1)

        fs, sel = _topk_f32(cs, TOPK)
        fi = jnp.take_along_axis(i0, sel >> 3, axis=1) \
            + jnp.take_along_axis(i1, sel & 7, axis=1)

        mx = jnp.max(fs, axis=-1, keepdims=True)
        ex = jnp.exp(fs - mx)
        rw = ex / jnp.sum(ex, axis=-1, keepdims=True)

        tt = TR // H
        hdr = jnp.broadcast_to(hd_ref[r0 // H:r0 // H + tt][:, None, :],
                               (tt, H, NE)).reshape(TR, NE)

        lo = jnp.take_along_axis(hdr[:, :HALF], jnp.minimum(fi, HALF - 1),
                                 axis=1)
        hi = jnp.take_along_axis(hdr[:, HALF:], jnp.maximum(fi - HALF, 0),
                                 axis=1)
        hval = jnp.where(fi < HALF, lo, hi)
        g = 0.5 * hval * (1.0 + jax.lax.erf(hval * 0.7071067811865476))
        g = g * rw

        iota_e = jax.lax.broadcasted_iota(jnp.int32, (TR, NE), 1)
        w = jnp.zeros((TR, NE), dtype=jnp.float32)
        for k in range(TOPK):
            eqk = fi[:, k:k + 1] == iota_e
            w = w + jnp.where(eqk, g[:, k:k + 1], 0.0)

        wt_parts.append(jnp.sum(w.reshape(tt, H, NE), axis=1))

    wt = jnp.concatenate(wt_parts, axis=0)
    out_ref[...] = jnp.dot(wt, up_ref[...],
                           preferred_element_type=jnp.float32)


def kernel(hidden_states, Wq, bq, bn_gamma, bn_beta, sub_keys_0, sub_keys_1,
           expert_down, expert_up):
    b, s, d = hidden_states.shape
    hs2 = hidden_states.reshape(S, D)
    dwn_t = expert_down[:NE].T
    up256 = expert_up[:NE]
    bq2 = bq.reshape(1, H * QD)

    q, hd, stats = pl.pallas_call(
        _qproj_kernel,
        grid=(S // T1,),
        in_specs=[
            pl.BlockSpec((T1, D), lambda i: (i, 0)),
            pl.BlockSpec((D, H * QD), lambda i: (0, 0)),
            pl.BlockSpec((1, H * QD), lambda i: (0, 0)),
            pl.BlockSpec((D, NE), lambda i: (0, 0)),
        ],
        out_specs=[
            pl.BlockSpec((T1, H * QD), lambda i: (i, 0)),
            pl.BlockSpec((T1, NE), lambda i: (i, 0)),
            pl.BlockSpec((2, H * QD), lambda i: (0, 0)),
        ],
        out_shape=[
            jax.ShapeDtypeStruct((S, H * QD), jnp.float32),
            jax.ShapeDtypeStruct((S, NE), jnp.float32),
            jax.ShapeDtypeStruct((2, H * QD), jnp.float32),
        ],
    )(hs2, Wq, bq2, dwn_t)

    n = float(S * H)
    ssum = stats[0].reshape(H, QD).sum(axis=0)
    ssq = stats[1].reshape(H, QD).sum(axis=0)
    mean = ssum / n
    var = ssq / n - mean * mean
    rstd = jax.lax.rsqrt(var + 1e-5)
    mean2 = mean.reshape(1, QD)
    rstd2 = rstd.reshape(1, QD)
    gam2 = bn_gamma.reshape(1, QD)
    bet2 = bn_beta.reshape(1, QD)
    q16 = q.reshape(S * H, QD)

    out = pl.pallas_call(
        _route_kernel,
        grid=(S // T2,),
        in_specs=[
            pl.BlockSpec((T2 * H, QD), lambda i: (i, 0)),
            pl.BlockSpec((T2, NE), lambda i: (i, 0)),
            pl.BlockSpec((1, QD), lambda i: (0, 0)),
            pl.BlockSpec((1, QD), lambda i: (0, 0)),
            pl.BlockSpec((1, QD), lambda i: (0, 0)),
            pl.BlockSpec((1, QD), lambda i: (0, 0)),
            pl.BlockSpec((HALF, HALF), lambda i: (0, 0)),
            pl.BlockSpec((HALF, HALF), lambda i: (0, 0)),
            pl.BlockSpec((NE, D), lambda i: (0, 0)),
        ],
        out_specs=pl.BlockSpec((T2, D), lambda i: (i, 0)),
        out_shape=jax.ShapeDtypeStruct((S, D), jnp.float32),
    )(q16, hd, mean2, rstd2, gam2, bet2, sub_keys_0, sub_keys_1, up256)

    return out.reshape(b, s, d)


# speedup vs baseline: 1.3514x; 1.3514x over previous
import jax
import jax.numpy as jnp
from jax.experimental import pallas as pl

S = 2048
D = 1024
H = 8
QD = 256
HALF = 128
NE = 256
TOPK = 8
KP = 8

T1 = 256
T2 = 128
TR = 256


def _qproj_kernel(hs_ref, wq_ref, bq_ref, dwn_ref, q_ref, hd_ref, stats_ref):
    i = pl.program_id(0)
    hs = hs_ref[...]
    q = jnp.dot(hs, wq_ref[...], preferred_element_type=jnp.float32) + bq_ref[...]
    q_ref[...] = q
    hd_ref[...] = jnp.dot(hs, dwn_ref[...], preferred_element_type=jnp.float32)
    s = jnp.sum(q, axis=0, keepdims=True)
    sq = jnp.sum(q * q, axis=0, keepdims=True)
    st = jnp.concatenate([s, sq], axis=0)

    @pl.when(i == 0)
    def _():
        stats_ref[...] = st

    @pl.when(i != 0)
    def _():
        stats_ref[...] = stats_ref[...] + st


def _l2rows(x):
    n = jnp.sqrt(jnp.sum(x * x, axis=-1, keepdims=True))
    return x / jnp.maximum(n, 1e-12)


def _topk_f32(vals, k):
    iota = jax.lax.broadcasted_iota(jnp.int32, vals.shape, 1)
    ss, ii = [], []
    for _ in range(k):
        m = jnp.max(vals, axis=-1, keepdims=True)
        idx = jnp.argmax(vals, axis=-1).astype(jnp.int32)[:, None]
        ss.append(m)
        ii.append(idx)
        vals = jnp.where(iota == idx, -1e30, vals)
    return jnp.concatenate(ss, axis=-1), jnp.concatenate(ii, axis=-1)


def _route_kernel(q_ref, hd_ref, mean_ref, rstd_ref, gam_ref, bet_ref,
                  sk0_ref, sk1_ref, up_ref, out_ref):
    R = q_ref.shape[0]
    qn = (q_ref[...] - mean_ref[...]) * rstd_ref[...] * gam_ref[...] \
        + bet_ref[...]
    qa = _l2rows(qn[:, :HALF])
    qb = _l2rows(qn[:, HALF:])
    sk0 = _l2rows(sk0_ref[...])
    sk1 = _l2rows(sk1_ref[...])
    dn = (((1,), (1,)), ((), ()))
    ds0f = jax.lax.dot_general(qa, sk0, dn, preferred_element_type=jnp.float32)
    ds1f = jax.lax.dot_general(qb, sk1, dn, preferred_element_type=jnp.float32)

    wt_parts = []
    for ti in range(R // TR):
        r0 = ti * TR
        ds0 = ds0f[r0:r0 + TR]
        ds1 = ds1f[r0:r0 + TR]

        s01, i01 = _topk_f32(jnp.concatenate([ds0, ds1], axis=0), KP)
        s0, s1 = s01[:TR], s01[TR:]
        i0, i1 = i01[:TR], i01[TR:]

        m64 = jax.lax.broadcasted_iota(jnp.int32, (TR, KP * KP), 1)
        cs = jnp.take_along_axis(s0, m64 >> 3, axis=1) \
            + jnp.take_along_axis(s1, m64 & 7, axis=1)

        fs, sel = _topk_f32(cs, TOPK)
        fi = jnp.take_along_axis(i0, sel >> 3, axis=1) \
            + jnp.take_along_axis(i1, sel & 7, axis=1)

        mx = jnp.max(fs, axis=-1, keepdims=True)
        ex = jnp.exp(fs - mx)
        rw = ex / jnp.sum(ex, axis=-1, keepdims=True)

        tt = TR // H
        hdr = jnp.broadcast_to(hd_ref[r0 // H:r0 // H + tt][:, None, :],
                               (tt, H, NE)).reshape(TR, NE)

        lo = jnp.take_along_axis(hdr[:, :HALF], jnp.minimum(fi, HALF - 1),
                                 axis=1)
        hi = jnp.take_along_axis(hdr[:, HALF:], jnp.maximum(fi - HALF, 0),
                                 axis=1)
        hval = jnp.where(fi < HALF, lo, hi)
        g = 0.5 * hval * (1.0 + jax.lax.erf(hval * 0.7071067811865476))
        g = g * rw

        iota_e = jax.lax.broadcasted_iota(jnp.int32, (TR, NE), 1)
        w = jnp.zeros((TR, NE), dtype=jnp.float32)
        for k in range(TOPK):
            eqk = fi[:, k:k + 1] == iota_e
            w = w + jnp.where(eqk, g[:, k:k + 1], 0.0)

        wt_parts.append(jnp.sum(w.reshape(tt, H, NE), axis=1))

    wt = jnp.concatenate(wt_parts, axis=0)
    out_ref[...] = jnp.dot(wt, up_ref[...],
                           preferred_element_type=jnp.float32)


def kernel(hidden_states, Wq, bq, bn_gamma, bn_beta, sub_keys_0, sub_keys_1,
           expert_down, expert_up):
    b, s, d = hidden_states.shape
    hs2 = hidden_states.reshape(S, D)
    dwn_t = expert_down[:NE].T
    up256 = expert_up[:NE]
    bq2 = bq.reshape(1, H * QD)

    q, hd, stats = pl.pallas_call(
        _qproj_kernel,
        grid=(S // T1,),
        in_specs=[
            pl.BlockSpec((T1, D), lambda i: (i, 0)),
            pl.BlockSpec((D, H * QD), lambda i: (0, 0)),
            pl.BlockSpec((1, H * QD), lambda i: (0, 0)),
            pl.BlockSpec((D, NE), lambda i: (0, 0)),
        ],
        out_specs=[
            pl.BlockSpec((T1, H * QD), lambda i: (i, 0)),
            pl.BlockSpec((T1, NE), lambda i: (i, 0)),
            pl.BlockSpec((2, H * QD), lambda i: (0, 0)),
        ],
        out_shape=[
            jax.ShapeDtypeStruct((S, H * QD), jnp.float32),
            jax.ShapeDtypeStruct((S, NE), jnp.float32),
            jax.ShapeDtypeStruct((2, H * QD), jnp.float32),
        ],
    )(hs2, Wq, bq2, dwn_t)

    n = float(S * H)
    ssum = stats[0].reshape(H, QD).sum(axis=0)
    ssq = stats[1].reshape(H, QD).sum(axis=0)
    mean = ssum / n
    var = ssq / n - mean * mean
    rstd = jax.lax.rsqrt(var + 1e-5)
    mean2 = mean.reshape(1, QD)
    rstd2 = rstd.reshape(1, QD)
    gam2 = bn_gamma.reshape(1, QD)
    bet2 = bn_beta.reshape(1, QD)
    q16 = q.reshape(S * H, QD)

    out = pl.pallas_call(
        _route_kernel,
        grid=(S // T2,),
        in_specs=[
            pl.BlockSpec((T2 * H, QD), lambda i: (i, 0)),
            pl.BlockSpec((T2, NE), lambda i: (i, 0)),
            pl.BlockSpec((1, QD), lambda i: (0, 0)),
            pl.BlockSpec((1, QD), lambda i: (0, 0)),
            pl.BlockSpec((1, QD), lambda i: (0, 0)),
            pl.BlockSpec((1, QD), lambda i: (0, 0)),
            pl.BlockSpec((HALF, HALF), lambda i: (0, 0)),
            pl.BlockSpec((HALF, HALF), lambda i: (0, 0)),
            pl.BlockSpec((NE, D), lambda i: (0, 0)),
        ],
        out_specs=pl.BlockSpec((T2, D), lambda i: (i, 0)),
        out_shape=jax.ShapeDtypeStruct((S, D), jnp.float32),
    )(q16, hd, mean2, rstd2, gam2, bet2, sub_keys_0, sub_keys_1, up256)

    return out.reshape(b, s, d)


# speedup vs baseline: 1.9474x; 1.4410x over previous
import jax
import jax.numpy as jnp
from jax.experimental import pallas as pl

S = 2048
D = 1024
H = 8
QD = 256
HALF = 128
NE = 256
TOPK = 8
KP = 8

T1 = 256
T2 = 128
TR = 1024


def _qproj_kernel(hs_ref, wq_ref, bq_ref, dwn_ref, q_ref, hd_ref, stats_ref):
    i = pl.program_id(0)
    hs = hs_ref[...]
    q = jnp.dot(hs, wq_ref[...], preferred_element_type=jnp.float32) + bq_ref[...]
    q_ref[...] = q
    hd_ref[...] = jnp.dot(hs, dwn_ref[...], preferred_element_type=jnp.float32)
    s = jnp.sum(q, axis=0, keepdims=True)
    sq = jnp.sum(q * q, axis=0, keepdims=True)
    st = jnp.concatenate([s, sq], axis=0)

    @pl.when(i == 0)
    def _():
        stats_ref[...] = st

    @pl.when(i != 0)
    def _():
        stats_ref[...] = stats_ref[...] + st


def _l2rows(x):
    n = jnp.sqrt(jnp.sum(x * x, axis=-1, keepdims=True))
    return x / jnp.maximum(n, 1e-12)


def _topk_f32(vals, k):
    iota = jax.lax.broadcasted_iota(jnp.int32, vals.shape, 1)
    orig = vals
    ii = []
    for _ in range(k):
        idx = jnp.argmax(vals, axis=-1).astype(jnp.int32)[:, None]
        ii.append(idx)
        vals = jnp.where(iota == idx, -1e30, vals)
    idxs = jnp.concatenate(ii, axis=-1)
    return jnp.take_along_axis(orig, idxs, axis=1), idxs


def _route_kernel(q_ref, hd_ref, mean_ref, rstd_ref, gam_ref, bet_ref,
                  sk0_ref, sk1_ref, up_ref, out_ref):
    R = q_ref.shape[0]
    qn = (q_ref[...] - mean_ref[...]) * rstd_ref[...] * gam_ref[...] \
        + bet_ref[...]
    qa = _l2rows(qn[:, :HALF])
    qb = _l2rows(qn[:, HALF:])
    sk0 = _l2rows(sk0_ref[...])
    sk1 = _l2rows(sk1_ref[...])
    dn = (((1,), (1,)), ((), ()))
    ds0f = jax.lax.dot_general(qa, sk0, dn, preferred_element_type=jnp.float32)
    ds1f = jax.lax.dot_general(qb, sk1, dn, preferred_element_type=jnp.float32)

    wt_parts = []
    for ti in range(R // TR):
        r0 = ti * TR
        ds0 = ds0f[r0:r0 + TR]
        ds1 = ds1f[r0:r0 + TR]

        s01, i01 = _topk_f32(jnp.concatenate([ds0, ds1], axis=0), KP)
        s0, s1 = s01[:TR], s01[TR:]
        i0, i1 = i01[:TR], i01[TR:]

        m64 = jax.lax.broadcasted_iota(jnp.int32, (TR, KP * KP), 1)
        cs = jnp.take_along_axis(s0, m64 >> 3, axis=1) \
            + jnp.take_along_axis(s1, m64 & 7, axis=1)

        fs, sel = _topk_f32(cs, TOPK)
        fi = jnp.take_along_axis(i0, sel >> 3, axis=1) \
            + jnp.take_along_axis(i1, sel & 7, axis=1)

        mx = jnp.max(fs, axis=-1, keepdims=True)
        ex = jnp.exp(fs - mx)
        rw = ex / jnp.sum(ex, axis=-1, keepdims=True)

        tt = TR // H
        hdr = jnp.broadcast_to(hd_ref[r0 // H:r0 // H + tt][:, None, :],
                               (tt, H, NE)).reshape(TR, NE)

        lo = jnp.take_along_axis(hdr[:, :HALF], jnp.minimum(fi, HALF - 1),
                                 axis=1)
        hi = jnp.take_along_axis(hdr[:, HALF:], jnp.maximum(fi - HALF, 0),
                                 axis=1)
        hval = jnp.where(fi < HALF, lo, hi)
        g = 0.5 * hval * (1.0 + jax.lax.erf(hval * 0.7071067811865476))
        g = g * rw

        iota_e = jax.lax.broadcasted_iota(jnp.int32, (TR, NE), 1)
        w = jnp.zeros((TR, NE), dtype=jnp.float32)
        for k in range(TOPK):
            eqk = fi[:, k:k + 1] == iota_e
            w = w + jnp.where(eqk, g[:, k:k + 1], 0.0)

        wt_parts.append(jnp.sum(w.reshape(tt, H, NE), axis=1))

    wt = jnp.concatenate(wt_parts, axis=0)
    out_ref[...] = jnp.dot(wt, up_ref[...],
                           preferred_element_type=jnp.float32)


def kernel(hidden_states, Wq, bq, bn_gamma, bn_beta, sub_keys_0, sub_keys_1,
           expert_down, expert_up):
    b, s, d = hidden_states.shape
    hs2 = hidden_states.reshape(S, D)
    dwn_t = expert_down[:NE].T
    up256 = expert_up[:NE]
    bq2 = bq.reshape(1, H * QD)

    q, hd, stats = pl.pallas_call(
        _qproj_kernel,
        grid=(S // T1,),
        in_specs=[
            pl.BlockSpec((T1, D), lambda i: (i, 0)),
            pl.BlockSpec((D, H * QD), lambda i: (0, 0)),
            pl.BlockSpec((1, H * QD), lambda i: (0, 0)),
            pl.BlockSpec((D, NE), lambda i: (0, 0)),
        ],
        out_specs=[
            pl.BlockSpec((T1, H * QD), lambda i: (i, 0)),
            pl.BlockSpec((T1, NE), lambda i: (i, 0)),
            pl.BlockSpec((2, H * QD), lambda i: (0, 0)),
        ],
        out_shape=[
            jax.ShapeDtypeStruct((S, H * QD), jnp.float32),
            jax.ShapeDtypeStruct((S, NE), jnp.float32),
            jax.ShapeDtypeStruct((2, H * QD), jnp.float32),
        ],
    )(hs2, Wq, bq2, dwn_t)

    n = float(S * H)
    ssum = stats[0].reshape(H, QD).sum(axis=0)
    ssq = stats[1].reshape(H, QD).sum(axis=0)
    mean = ssum / n
    var = ssq / n - mean * mean
    rstd = jax.lax.rsqrt(var + 1e-5)
    mean2 = mean.reshape(1, QD)
    rstd2 = rstd.reshape(1, QD)
    gam2 = bn_gamma.reshape(1, QD)
    bet2 = bn_beta.reshape(1, QD)
    q16 = q.reshape(S * H, QD)

    out = pl.pallas_call(
        _route_kernel,
        grid=(S // T2,),
        in_specs=[
            pl.BlockSpec((T2 * H, QD), lambda i: (i, 0)),
            pl.BlockSpec((T2, NE), lambda i: (i, 0)),
            pl.BlockSpec((1, QD), lambda i: (0, 0)),
            pl.BlockSpec((1, QD), lambda i: (0, 0)),
            pl.BlockSpec((1, QD), lambda i: (0, 0)),
            pl.BlockSpec((1, QD), lambda i: (0, 0)),
            pl.BlockSpec((HALF, HALF), lambda i: (0, 0)),
            pl.BlockSpec((HALF, HALF), lambda i: (0, 0)),
            pl.BlockSpec((NE, D), lambda i: (0, 0)),
        ],
        out_specs=pl.BlockSpec((T2, D), lambda i: (i, 0)),
        out_shape=jax.ShapeDtypeStruct((S, D), jnp.float32),
    )(q16, hd, mean2, rstd2, gam2, bet2, sub_keys_0, sub_keys_1, up256)

    return out.reshape(b, s, d)
